# Initial kernel scaffold; baseline (speedup 1.0000x reference)
import jax
import jax.numpy as jnp
from jax.experimental import pallas as pl


def kernel(x, E):
    raise NotImplementedError("write your pallas kernel here")


# speedup vs baseline: 4.8069x; 4.8069x over previous
import functools

import jax
import jax.numpy as jnp
from jax import lax
from jax.experimental import pallas as pl
from jax.experimental.pallas import tpu as pltpu
from jax.experimental.pallas import tpu_sc as plsc

D = 32
NC, NS = 2, 16
NW = NC * NS
IDXW = 128
K = 8
BLK = K * IDXW


@functools.lru_cache(maxsize=None)
def _make_gather(R: int):
    RW = R // NW
    NB = RW // BLK
    mesh = plsc.VectorSubcoreMesh(core_axis_name="c", subcore_axis_name="s")

    @functools.partial(
        pl.kernel,
        out_type=jax.ShapeDtypeStruct((R, D), jnp.float32),
        mesh=mesh,
        scratch_types=[
            pltpu.VMEM((K, IDXW), jnp.int32),
            pltpu.VMEM((BLK, D), jnp.float32),
            pltpu.SemaphoreType.DMA,
        ],
        compiler_params=pltpu.CompilerParams(use_tc_tiling_on_sc=False),
    )
    def body(idx_hbm, tab_hbm, out_hbm, idx_v, rows_v, gsem):
        wid = lax.axis_index("s") * NC + lax.axis_index("c")
        idx_row0 = wid * (RW // IDXW)
        out_row0 = wid * RW

        @pl.loop(0, NB)
        def _block(g):
            pltpu.sync_copy(idx_hbm.at[pl.ds(idx_row0 + g * K, K)], idx_v)
            copies = [
                pltpu.async_copy(
                    tab_hbm.at[idx_v.at[j]],
                    rows_v.at[pl.ds(j * IDXW, IDXW)],
                    gsem,
                )
                for j in range(K)
            ]
            for c in copies:
                c.wait()
            pltpu.sync_copy(rows_v, out_hbm.at[pl.ds(out_row0 + g * BLK, BLK)])

    return body


def kernel(x, E):
    B, H = x.shape
    R = B * H
    idx = x.astype(jnp.int32).reshape(R // IDXW, IDXW)
    out = _make_gather(R)(idx, E)
    return out.reshape(B, H, D)


# speedup vs baseline: 4.9508x; 1.0299x over previous
import functools

import jax
import jax.numpy as jnp
from jax import lax
from jax.experimental import pallas as pl
from jax.experimental.pallas import tpu as pltpu
from jax.experimental.pallas import tpu_sc as plsc

D = 32
NC, NS = 2, 16
NW = NC * NS
IDXW = 128
K = 8
BLK = K * IDXW


@functools.lru_cache(maxsize=None)
def _make_gather(R: int):
    RW = R // NW
    NB = RW // BLK
    mesh = plsc.VectorSubcoreMesh(core_axis_name="c", subcore_axis_name="s")

    assert NB % 2 == 0 and NB >= 4

    @functools.partial(
        pl.kernel,
        out_type=jax.ShapeDtypeStruct((R, D), jnp.float32),
        mesh=mesh,
        scratch_types=[
            pltpu.VMEM((2, K, IDXW), jnp.int32),
            pltpu.VMEM((2, BLK, D), jnp.float32),
            pltpu.SemaphoreType.DMA,
            pltpu.SemaphoreType.DMA,
        ],
        compiler_params=pltpu.CompilerParams(use_tc_tiling_on_sc=False),
    )
    def body(idx_hbm, tab_hbm, out_hbm, idx_v, rows_v, gsem, osem):
        wid = lax.axis_index("s") * NC + lax.axis_index("c")
        idx_row0 = wid * (RW // IDXW)
        out_row0 = wid * RW

        def fire_gathers(g, b):
            pltpu.sync_copy(idx_hbm.at[pl.ds(idx_row0 + g * K, K)],
                            idx_v.at[b])
            for j in range(K):
                pltpu.async_copy(
                    tab_hbm.at[idx_v.at[b, j]],
                    rows_v.at[b, pl.ds(j * IDXW, IDXW)],
                    gsem,
                )

        def drain_gathers(b):
            for j in range(K):
                pltpu.make_async_copy(
                    tab_hbm.at[idx_v.at[b, j]],
                    rows_v.at[b, pl.ds(j * IDXW, IDXW)],
                    gsem,
                ).wait()

        def store(g, b):
            pltpu.async_copy(rows_v.at[b],
                             out_hbm.at[pl.ds(out_row0 + g * BLK, BLK)],
                             osem).wait()

        fire_gathers(0, 0)
        fire_gathers(1, 1)

        @pl.loop(0, NB - 2, step=2)
        def _pair(g0):
            for b in range(2):
                g = g0 + b
                drain_gathers(b)
                store(g, b)
                fire_gathers(g + 2, b)

        for b in range(2):
            drain_gathers(b)
            store(NB - 2 + b, b)

    return body


def kernel(x, E):
    B, H = x.shape
    R = B * H
    idx = x.astype(jnp.int32).reshape(R // IDXW, IDXW)
    out = _make_gather(R)(idx, E)
    return out.reshape(B, H, D)
